# baseline (speedup 1.0000x reference)
import jax
import jax.numpy as jnp
from jax.experimental import pallas as pl
from jax.experimental.pallas import tpu as pltpu

_T = 2048


def _median5(z0, z1, z2, z3, z4):
    a0 = jnp.minimum(z0, z1)
    a1 = jnp.maximum(z0, z1)
    a3 = jnp.minimum(z3, z4)
    a4 = jnp.maximum(z3, z4)
    b3 = jnp.maximum(a0, a3)
    b1 = jnp.minimum(a1, a4)
    c1 = jnp.minimum(b1, z2)
    c2 = jnp.maximum(b1, z2)
    d2 = jnp.minimum(c2, b3)
    return jnp.maximum(c1, d2)


def _medpool(v):
    T = _T
    s0 = jnp.concatenate([v[:, 2:3], v[:, 1:2], v[:, :T - 2]], axis=1)
    s1 = jnp.concatenate([v[:, 1:2], v[:, :T - 1]], axis=1)
    s3 = jnp.concatenate([v[:, 1:], v[:, T - 2:T - 1]], axis=1)
    s4 = jnp.concatenate([v[:, 2:], v[:, T - 2:T - 1], v[:, T - 3:T - 2]],
                         axis=1)
    return _median5(s0, s1, v, s3, s4)


def _fused_body(x_ref, u_ref, w_ref, b_ref, e_ref, post_ref, mask_ref):
    xb = x_ref[0]
    w2 = w_ref[...]
    h = jax.lax.dot_general(w2, xb, (((1,), (0,)), ((), ())),
                            preferred_element_type=jnp.float32)
    ut = jnp.transpose(u_ref[0], (1, 0))
    z0 = (h[0:1, :] + b_ref[0]) / 10.0
    z1 = (h[1:2, :] + b_ref[1]) / 10.0
    m = jnp.maximum(z0, z1)
    e0 = jnp.exp(z0 - m)
    e1 = jnp.exp(z1 - m)
    s = e0 + e1
    p0 = e0 / s
    p1 = e1 / s
    post_ref[0] = jnp.transpose(jnp.concatenate([p0, p1], axis=0), (1, 0))
    eps = 1e-20
    l0 = jnp.log(p0)
    l1 = jnp.log(p1)
    u0 = ut[0:1, :]
    u1 = ut[1:2, :]
    g0 = -jnp.log(-jnp.log(u0 + eps) + eps)
    g1 = -jnp.log(-jnp.log(u1 + eps) + eps)
    zz0 = (l0 + g0) / 0.8
    zz1 = (l1 + g1) / 0.8
    mm = jnp.maximum(zz0, zz1)
    ee0 = jnp.exp(zz0 - mm)
    ee1 = jnp.exp(zz1 - mm)
    ss = ee0 + ee1
    y0 = ee0 / ss
    y1 = ee1 / ss
    selv = jnp.where(y1 > y0, 1.0, 0.0).astype(jnp.bfloat16)
    v = e_ref[0].astype(jnp.bfloat16) * selv
    v = _medpool(v)
    v = _medpool(v)
    v = _medpool(v)
    mask_ref[0] = v.astype(jnp.float32)


def kernel(x, e, u, W, b):
    B, C, T = x.shape

    posterior, mask = pl.pallas_call(
        _fused_body,
        grid=(B,),
        in_specs=[
            pl.BlockSpec((1, C, T), lambda i: (i, 0, 0)),
            pl.BlockSpec((1, T, 2), lambda i: (i, 0, 0)),
            pl.BlockSpec((2, C), lambda i: (0, 0)),
            pl.BlockSpec(memory_space=pltpu.SMEM),
            pl.BlockSpec((1, C, T), lambda i: (i, 0, 0)),
        ],
        out_specs=[
            pl.BlockSpec((1, T, 2), lambda i: (i, 0, 0)),
            pl.BlockSpec((1, C, T), lambda i: (i, 0, 0)),
        ],
        out_shape=[
            jax.ShapeDtypeStruct((B, T, 2), jnp.float32),
            jax.ShapeDtypeStruct((B, C, T), jnp.float32),
        ],
    )(x, u, W, b, e)

    return posterior, mask


# speedup vs baseline: 1.2117x; 1.2117x over previous
import jax
import jax.numpy as jnp
from jax.experimental import pallas as pl
from jax.experimental.pallas import tpu as pltpu

_T = 2048


def _median5(z0, z1, z2, z3, z4):
    a0 = jnp.minimum(z0, z1)
    a1 = jnp.maximum(z0, z1)
    a3 = jnp.minimum(z3, z4)
    a4 = jnp.maximum(z3, z4)
    b3 = jnp.maximum(a0, a3)
    b1 = jnp.minimum(a1, a4)
    c1 = jnp.minimum(b1, z2)
    c2 = jnp.maximum(b1, z2)
    d2 = jnp.minimum(c2, b3)
    return jnp.maximum(c1, d2)


def _medpool(v):
    T = _T
    s0 = jnp.concatenate([v[:, 2:3], v[:, 1:2], v[:, :T - 2]], axis=1)
    s1 = jnp.concatenate([v[:, 1:2], v[:, :T - 1]], axis=1)
    s3 = jnp.concatenate([v[:, 1:], v[:, T - 2:T - 1]], axis=1)
    s4 = jnp.concatenate([v[:, 2:], v[:, T - 2:T - 1], v[:, T - 3:T - 2]],
                         axis=1)
    return _median5(s0, s1, v, s3, s4)


def _fused_body(x_ref, ut_ref, w_ref, b_ref, e_ref, post_ref, mask_ref):
    xb = x_ref[0]
    w2 = w_ref[...]
    h = jax.lax.dot_general(w2, xb, (((1,), (0,)), ((), ())),
                            preferred_element_type=jnp.float32)
    z0 = (h[0:1, :] + b_ref[0]) / 10.0
    z1 = (h[1:2, :] + b_ref[1]) / 10.0
    m = jnp.maximum(z0, z1)
    e0 = jnp.exp(z0 - m)
    e1 = jnp.exp(z1 - m)
    s = e0 + e1
    p0 = e0 / s
    p1 = e1 / s
    post_ref[0, 0:1, :] = p0
    post_ref[0, 1:2, :] = p1
    eps = 1e-20
    l0 = jnp.log(p0)
    l1 = jnp.log(p1)
    u0 = ut_ref[0, 0:1, :]
    u1 = ut_ref[0, 1:2, :]
    g0 = -jnp.log(-jnp.log(u0 + eps) + eps)
    g1 = -jnp.log(-jnp.log(u1 + eps) + eps)
    zz0 = (l0 + g0) / 0.8
    zz1 = (l1 + g1) / 0.8
    mm = jnp.maximum(zz0, zz1)
    ee0 = jnp.exp(zz0 - mm)
    ee1 = jnp.exp(zz1 - mm)
    ss = ee0 + ee1
    y0 = ee0 / ss
    y1 = ee1 / ss
    selv = jnp.where(y1 > y0, 1.0, 0.0).astype(jnp.bfloat16)
    v = e_ref[0].astype(jnp.bfloat16) * selv
    v = _medpool(v)
    v = _medpool(v)
    v = _medpool(v)
    mask_ref[0] = v.astype(jnp.float32)


def kernel(x, e, u, W, b):
    B, C, T = x.shape
    ut = jnp.transpose(u, (0, 2, 1))

    post_bt, mask = pl.pallas_call(
        _fused_body,
        grid=(B,),
        in_specs=[
            pl.BlockSpec((1, C, T), lambda i: (i, 0, 0)),
            pl.BlockSpec((1, 2, T), lambda i: (i, 0, 0)),
            pl.BlockSpec((2, C), lambda i: (0, 0)),
            pl.BlockSpec(memory_space=pltpu.SMEM),
            pl.BlockSpec((1, C, T), lambda i: (i, 0, 0)),
        ],
        out_specs=[
            pl.BlockSpec((1, 2, T), lambda i: (i, 0, 0)),
            pl.BlockSpec((1, C, T), lambda i: (i, 0, 0)),
        ],
        out_shape=[
            jax.ShapeDtypeStruct((B, 2, T), jnp.float32),
            jax.ShapeDtypeStruct((B, C, T), jnp.float32),
        ],
    )(x, ut, W, b, e)

    posterior = jnp.transpose(post_bt, (0, 2, 1))
    return posterior, mask
